# Initial kernel scaffold; baseline (speedup 1.0000x reference)
import jax
import jax.numpy as jnp
from jax.experimental import pallas as pl


def kernel(x, edge_index, target_tasklets, target_map_entry, W_msg, W_out, b_out, Wt1, bt1, Wt2, bt2, Wm1, bm1, Wm2, bm2):
    raise NotImplementedError("write your pallas kernel here")


# speedup vs baseline: 1.7166x; 1.7166x over previous
import jax
import jax.numpy as jnp
from jax import lax
from jax.experimental import pallas as pl
from jax.experimental.pallas import tpu as pltpu
from jax.experimental.pallas import tpu_sc as plsc

N = 50000
E = 800000
D = 70
K = 4
T = 2048

NC, NS = 2, 16
NW = NC * NS
DP = 128
SN = 1568
NP = NW * SN
U = 2 * T
SW = U // NW
CH = 2048
E_PAD = 800768
NCHUNK = E_PAD // CH
LCAP = 2048
LBUF = LCAP + 16
OP = 128

_MESH = plsc.VectorSubcoreMesh(core_axis_name="c", subcore_axis_name="s",
                               num_cores=NC, num_subcores=NS)
_INT_MIN = -(2 ** 31)


def _wid():
    return lax.axis_index("s") * NC + lax.axis_index("c")


def _mm_body(x_ref, w_ref, o_ref):
    o_ref[...] = jnp.dot(x_ref[...], w_ref[...],
                         preferred_element_type=jnp.float32)


def _msg_matmul(x_p, w_p):
    BM = 512
    return pl.pallas_call(
        _mm_body,
        grid=(NP // BM,),
        in_specs=[pl.BlockSpec((BM, D), lambda i: (i, 0)),
                  pl.BlockSpec((D, DP), lambda i: (0, 0))],
        out_specs=pl.BlockSpec((BM, DP), lambda i: (i, 0)),
        out_shape=jax.ShapeDtypeStruct((NP, DP), jnp.float32),
    )(x_p, w_p)


def _dmap_body(tgt_hbm, dmap_hbm, tgt_v, loc_v):
    lo = _wid() * SN
    neg1 = jnp.full((16,), -1, jnp.int32)

    def init(i, _):
        loc_v[pl.ds(i * 16, 16)] = neg1
        return 0
    lax.fori_loop(0, SN // 16, init, 0)

    pltpu.sync_copy(tgt_hbm, tgt_v)
    lane = lax.iota(jnp.int32, 16)

    def scat(g, _):
        t = tgt_v[pl.ds(g * 16, 16)]
        m = (t >= lo) & (t < lo + SN)
        idx = jnp.where(m, t - lo, 0)
        plsc.store_scatter(loc_v, [idx], g * 16 + lane, mask=m)
        return 0
    lax.fori_loop(0, U // 16, scat, 0)

    pltpu.sync_copy(loc_v, dmap_hbm.at[pl.ds(lo, SN)])


def _build_dmap(tgt):
    return pl.kernel(
        _dmap_body,
        out_type=jax.ShapeDtypeStruct((NP,), jnp.int32),
        mesh=_MESH,
        compiler_params=pltpu.CompilerParams(needs_layout_passes=False),
        scratch_types=[pltpu.VMEM((U,), jnp.int32),
                       pltpu.VMEM((SN,), jnp.int32)],
    )(tgt)


def _agg_body(dst_hbm, src_hbm, dmap_hbm, y_hbm, agg_hbm,
              dmap_v, dstb, srcb, lslot, lsrc, rows, acc, sem):
    lo = _wid() * SW
    pltpu.sync_copy(dmap_hbm, dmap_v)

    ninf = jnp.full((16,), -jnp.inf, jnp.float32)

    def init_acc(i, _):
        acc[pl.ds(i * 16, 16)] = ninf
        return 0
    lax.fori_loop(0, SW * DP // 16, init_acc, 0)

    zero16 = jnp.zeros((16,), jnp.int32)

    def init_src(i, _):
        lsrc[pl.ds(i * 16, 16)] = zero16
        return 0
    lax.fori_loop(0, LBUF // 16, init_src, 0)

    lane = lax.iota(jnp.int32, 16)

    def drain(n):
        def q_body(q, _):
            sv = lsrc[pl.ds(q * 16, 16)]
            tv = lslot[pl.ds(q * 16, 16)]
            pltpu.async_copy(y_hbm.at[sv], rows, sem).wait()
            for l in range(16):
                @pl.when(q * 16 + l < n)
                def _():
                    s = jnp.max(jnp.where(lane == l, tv, _INT_MIN)) - lo
                    b = s * DP
                    for c in range(DP // 16):
                        av = acc[pl.ds(b + c * 16, 16)]
                        rv = rows[l, pl.ds(c * 16, 16)]
                        acc[pl.ds(b + c * 16, 16)] = jnp.maximum(av, rv)
            return 0
        lax.fori_loop(0, (n + 15) // 16, q_body, 0)

    def chunk(ci, cnt):
        pltpu.sync_copy(dst_hbm.at[pl.ds(ci * CH, CH)], dstb)
        pltpu.sync_copy(src_hbm.at[pl.ds(ci * CH, CH)], srcb)

        def grp(g, cnt):
            dv = dstb[pl.ds(g * 16, 16)]
            sl = plsc.load_gather(dmap_v, [dv])
            m = (sl >= lo) & (sl < lo + SW)
            plsc.store_compressed(lslot.at[pl.ds(cnt, 16)], sl, mask=m)
            plsc.store_compressed(lsrc.at[pl.ds(cnt, 16)],
                                  srcb[pl.ds(g * 16, 16)], mask=m)
            cnt = cnt + jnp.max(plsc.all_reduce_population_count(m))
            full = cnt > LCAP - 16

            @pl.when(full)
            def _():
                drain(cnt)
            return jnp.where(full, 0, cnt)
        return lax.fori_loop(0, CH // 16, grp, cnt)

    cnt = lax.fori_loop(0, NCHUNK, chunk, jnp.int32(0))
    drain(cnt)
    pltpu.sync_copy(acc, agg_hbm.at[pl.ds(lo * DP, SW * DP)])


def _scatter_max(dst_p, src_p, dmap, y):
    return pl.kernel(
        _agg_body,
        out_type=jax.ShapeDtypeStruct((U * DP,), jnp.float32),
        mesh=_MESH,
        compiler_params=pltpu.CompilerParams(needs_layout_passes=False),
        scratch_types=[pltpu.VMEM((NP,), jnp.int32),
                       pltpu.VMEM((CH,), jnp.int32),
                       pltpu.VMEM((CH,), jnp.int32),
                       pltpu.VMEM((LBUF,), jnp.int32),
                       pltpu.VMEM((LBUF,), jnp.int32),
                       pltpu.VMEM((16, DP), jnp.float32),
                       pltpu.VMEM((SW * DP,), jnp.float32),
                       pltpu.SemaphoreType.DMA],
    )(dst_p, src_p, dmap, y)


def _tgt_body(tgt_hbm, dmap_hbm, agg_hbm, out_hbm, tgt_v, dmap_v, rows, sem):
    nt = U // NW
    base = _wid() * nt
    pltpu.sync_copy(tgt_hbm.at[pl.ds(base, nt)], tgt_v)
    pltpu.sync_copy(dmap_hbm, dmap_v)
    for q in range(nt // 16):
        t = tgt_v[pl.ds(q * 16, 16)]
        sl = plsc.load_gather(dmap_v, [t])
        pltpu.async_copy(agg_hbm.at[sl], rows, sem).wait()
        pltpu.sync_copy(rows, out_hbm.at[pl.ds(base + q * 16, 16)])


def _gather_targets(tgt, dmap, agg2):
    return pl.kernel(
        _tgt_body,
        out_type=jax.ShapeDtypeStruct((U, DP), jnp.float32),
        mesh=_MESH,
        compiler_params=pltpu.CompilerParams(needs_layout_passes=False),
        scratch_types=[pltpu.VMEM((U // NW,), jnp.int32),
                       pltpu.VMEM((NP,), jnp.int32),
                       pltpu.VMEM((16, DP), jnp.float32),
                       pltpu.SemaphoreType.DMA],
    )(tgt, dmap, agg2)


def _heads_body(rt_ref, rm_ref, wo_ref, bo_ref,
                wt1_ref, bt1_ref, wt2_ref, bt2_ref,
                wm1_ref, bm1_ref, wm2_ref, bm2_ref,
                ot_ref, om_ref):
    wo = wo_ref[...]
    bo = bo_ref[...]

    def group(a_ref, w1_ref, b1_ref, w2_ref, b2_ref, o_ref):
        a = a_ref[...][:, :D]
        a = jnp.where(jnp.isfinite(a), a, 0.0)
        r = jnp.dot(a, wo, preferred_element_type=jnp.float32) + bo
        for k in range(K):
            h = jnp.maximum(
                jnp.dot(r, w1_ref[k], preferred_element_type=jnp.float32)
                + b1_ref[...][k][None, :], 0.0)
            o_ref[k] = (jnp.dot(h, w2_ref[k], preferred_element_type=jnp.float32)
                        + b2_ref[...][k][None, :])

    group(rt_ref, wt1_ref, bt1_ref, wt2_ref, bt2_ref, ot_ref)
    group(rm_ref, wm1_ref, bm1_ref, wm2_ref, bm2_ref, om_ref)


def _heads(rt_agg, rm_agg, W_out, b_out2, Wt1, bt1, Wt2p, bt2p,
           Wm1, bm1, Wm2p, bm2p):
    return pl.pallas_call(
        _heads_body,
        out_shape=[jax.ShapeDtypeStruct((K, T, OP), jnp.float32),
                   jax.ShapeDtypeStruct((K, T, OP), jnp.float32)],
    )(rt_agg, rm_agg, W_out, b_out2, Wt1, bt1, Wt2p, bt2p,
      Wm1, bm1, Wm2p, bm2p)


def kernel(x, edge_index, target_tasklets, target_map_entry,
           W_msg, W_out, b_out, Wt1, bt1, Wt2, bt2, Wm1, bm1, Wm2, bm2):
    f32 = jnp.float32
    x_p = jnp.zeros((NP, D), f32).at[:N, :].set(x)
    w_p = jnp.zeros((D, DP), f32).at[:, :D].set(W_msg)
    y = _msg_matmul(x_p, w_p)

    tgt = jnp.concatenate([target_tasklets[0], target_map_entry[0]])
    dmap = _build_dmap(tgt)

    src_p = jnp.zeros((E_PAD,), jnp.int32).at[:E].set(edge_index[0])
    dst_p = jnp.full((E_PAD,), N, jnp.int32).at[:E].set(edge_index[1])
    agg_flat = _scatter_max(dst_p, src_p, dmap, y)
    agg2 = agg_flat.reshape(U, DP)

    rtm = _gather_targets(tgt, dmap, agg2)
    rt_agg, rm_agg = rtm[:T], rtm[T:]

    b_out2 = b_out.reshape(1, D)
    Wt2p = jnp.zeros((K, D, OP), f32).at[:, :, :2].set(Wt2)
    bt2p = jnp.zeros((K, OP), f32).at[:, :2].set(bt2)
    Wm2p = jnp.zeros((K, D, OP), f32).at[:, :, :2].set(Wm2)
    bm2p = jnp.zeros((K, OP), f32).at[:, :2].set(bm2)
    ot, om = _heads(rt_agg, rm_agg, W_out, b_out2, Wt1, bt1, Wt2p, bt2p,
                    Wm1, bm1, Wm2p, bm2p)
    return ot[:, :, :2], om[:, :, :2]


# speedup vs baseline: 9.6081x; 5.5972x over previous
import jax
import jax.numpy as jnp
from jax import lax
from jax.experimental import pallas as pl
from jax.experimental.pallas import tpu as pltpu
from jax.experimental.pallas import tpu_sc as plsc

N = 50000
E = 800000
D = 70
K = 4
T = 2048

NC, NS = 2, 16
NW = NC * NS
DP = 128
SN = 1568
NP = NW * SN
U = 2 * T
SW = U // NW
CH = 2048

EW1 = 25024
NFULL0, TAIL0 = 12, 448
NFULL1, TAIL1 = 11, 1728
CROW = 13 * CH
CBUF = CROW + 16

LCAP = 2048
LBUF = LCAP + 16
OP = 128

_MESH = plsc.VectorSubcoreMesh(core_axis_name="c", subcore_axis_name="s",
                               num_cores=NC, num_subcores=NS)
_SC_PARAMS = pltpu.CompilerParams(needs_layout_passes=False)


def _wid():
    return lax.axis_index("s") * NC + lax.axis_index("c")


def _mm_body(x_ref, w_ref, o_ref):
    o_ref[...] = jnp.dot(x_ref[...], w_ref[...],
                         preferred_element_type=jnp.float32)


def _msg_matmul(x, w_p):
    BM = 512
    return pl.pallas_call(
        _mm_body,
        grid=(pl.cdiv(N, BM),),
        in_specs=[pl.BlockSpec((BM, D), lambda i: (i, 0)),
                  pl.BlockSpec((D, DP), lambda i: (0, 0))],
        out_specs=pl.BlockSpec((BM, DP), lambda i: (i, 0)),
        out_shape=jax.ShapeDtypeStruct((N, DP), jnp.float32),
    )(x, w_p)


def _dmap_body(tgt_hbm, dmap_hbm, tgt_v, loc_v):
    lo = _wid() * SN
    neg1 = jnp.full((16,), -1, jnp.int32)

    def init(i, _):
        loc_v[pl.ds(i * 16, 16)] = neg1
        return 0
    lax.fori_loop(0, SN // 16, init, 0)

    pltpu.sync_copy(tgt_hbm, tgt_v)
    lane = lax.iota(jnp.int32, 16)

    def scat(g, _):
        t = tgt_v[pl.ds(g * 16, 16)]
        m = (t >= lo) & (t < lo + SN)
        idx = jnp.where(m, t - lo, 0)
        plsc.store_scatter(loc_v, [idx], g * 16 + lane, mask=m)
        return 0
    lax.fori_loop(0, U // 16, scat, 0)

    pltpu.sync_copy(loc_v, dmap_hbm.at[pl.ds(lo, SN)])


def _build_dmap(tgt):
    return pl.kernel(
        _dmap_body,
        out_type=jax.ShapeDtypeStruct((NP,), jnp.int32),
        mesh=_MESH,
        compiler_params=_SC_PARAMS,
        scratch_types=[pltpu.VMEM((U,), jnp.int32),
                       pltpu.VMEM((SN,), jnp.int32)],
    )(tgt)


def _pre_body(dst_hbm, src_hbm, dmap_hbm, cslot_hbm, csrc_hbm, cnts_hbm,
              dmap_v, dstb, srcb, lslot, lsrc, cstage):
    w = _wid()
    pltpu.sync_copy(dmap_hbm, dmap_v)
    ebase = w * EW1

    def scan_groups(ngrp, cnt):
        def grp(g, cnt):
            dv = dstb[pl.ds(g * 16, 16)]
            sl = plsc.load_gather(dmap_v, [dv])
            m = sl >= 0
            plsc.store_compressed(lslot.at[pl.ds(cnt, 16)], sl, mask=m)
            plsc.store_compressed(lsrc.at[pl.ds(cnt, 16)],
                                  srcb[pl.ds(g * 16, 16)], mask=m)
            return cnt + plsc.all_reduce_population_count(m)[0]
        return lax.fori_loop(0, ngrp, grp, cnt)

    nfull = jnp.where(w < NW - 1, NFULL0, NFULL1)

    def chunk(ci, cnt):
        pltpu.sync_copy(dst_hbm.at[pl.ds(ebase + ci * CH, CH)], dstb)
        pltpu.sync_copy(src_hbm.at[pl.ds(ebase + ci * CH, CH)], srcb)
        return scan_groups(CH // 16, cnt)

    cnt = lax.fori_loop(0, nfull, chunk, jnp.int32(0))

    @pl.when(w < NW - 1)
    def _():
        pltpu.sync_copy(dst_hbm.at[pl.ds(ebase + NFULL0 * CH, TAIL0)],
                        dstb.at[pl.ds(0, TAIL0)])
        pltpu.sync_copy(src_hbm.at[pl.ds(ebase + NFULL0 * CH, TAIL0)],
                        srcb.at[pl.ds(0, TAIL0)])

    @pl.when(w == NW - 1)
    def _():
        pltpu.sync_copy(dst_hbm.at[pl.ds(ebase + NFULL1 * CH, TAIL1)],
                        dstb.at[pl.ds(0, TAIL1)])
        pltpu.sync_copy(src_hbm.at[pl.ds(ebase + NFULL1 * CH, TAIL1)],
                        srcb.at[pl.ds(0, TAIL1)])

    ntailg = jnp.where(w < NW - 1, TAIL0 // 16, TAIL1 // 16)
    cnt = scan_groups(ntailg, cnt)

    roundup = ((cnt + CH - 1) // CH) * CH
    neg1 = jnp.full((16,), -1, jnp.int32)

    def fill(j, _):
        lslot[pl.ds(cnt + j * 16, 16)] = neg1
        return 0
    lax.fori_loop(0, (roundup - cnt + 15) // 16, fill, 0)

    pltpu.sync_copy(lslot.at[pl.ds(0, CROW)],
                    cslot_hbm.at[pl.ds(w * CROW, CROW)])
    pltpu.sync_copy(lsrc.at[pl.ds(0, CROW)],
                    csrc_hbm.at[pl.ds(w * CROW, CROW)])
    cstage[pl.ds(0, 16)] = jnp.zeros((16,), jnp.int32) + cnt
    pltpu.sync_copy(cstage, cnts_hbm.at[pl.ds(w * 16, 16)])


def _prefilter(dst, src, dmap):
    return pl.kernel(
        _pre_body,
        out_type=[jax.ShapeDtypeStruct((NW * CROW,), jnp.int32),
                  jax.ShapeDtypeStruct((NW * CROW,), jnp.int32),
                  jax.ShapeDtypeStruct((NW * 16,), jnp.int32)],
        mesh=_MESH,
        compiler_params=_SC_PARAMS,
        scratch_types=[pltpu.VMEM((NP,), jnp.int32),
                       pltpu.VMEM((CH,), jnp.int32),
                       pltpu.VMEM((CH,), jnp.int32),
                       pltpu.VMEM((CBUF,), jnp.int32),
                       pltpu.VMEM((CBUF,), jnp.int32),
                       pltpu.VMEM((16,), jnp.int32)],
    )(dst, src, dmap)


def _merge_body(cslot_hbm, csrc_hbm, cnts_hbm, y_hbm, agg_hbm,
                cntsv, slotb, srcb, lslot, lsrc, rows, acc, sem):
    lo = _wid() * SW
    pltpu.sync_copy(cnts_hbm, cntsv)

    ninf = jnp.full((16,), -jnp.inf, jnp.float32)

    def init_acc(i, _):
        acc[pl.ds(i * 16, 16)] = ninf
        return 0
    lax.fori_loop(0, SW * DP // 16, init_acc, 0)

    zero16 = jnp.zeros((16,), jnp.int32)

    def init_src(i, _):
        lsrc[pl.ds(i * 16, 16)] = zero16
        return 0
    lax.fori_loop(0, LBUF // 16, init_src, 0)

    def drain(n):
        def q_body(q, _):
            sv = lsrc[pl.ds(q * 16, 16)]
            tv = lslot[pl.ds(q * 16, 16)]
            pltpu.async_copy(y_hbm.at[sv], rows, sem).wait()
            for l in range(16):
                @pl.when(q * 16 + l < n)
                def _():
                    b = (tv[l] - lo) * DP
                    for c in range(DP // 16):
                        av = acc[pl.ds(b + c * 16, 16)]
                        rv = rows[l, pl.ds(c * 16, 16)]
                        acc[pl.ds(b + c * 16, 16)] = jnp.maximum(av, rv)
            return 0
        lax.fori_loop(0, (n + 15) // 16, q_body, 0)

    def one_list(v, cnt):
        nch = (cntsv[pl.ds(v * 16, 16)][0] + CH - 1) // CH

        def chunk(ci, cnt):
            pltpu.sync_copy(cslot_hbm.at[pl.ds(v * CROW + ci * CH, CH)], slotb)
            pltpu.sync_copy(csrc_hbm.at[pl.ds(v * CROW + ci * CH, CH)], srcb)

            def grp(g, cnt):
                sl = slotb[pl.ds(g * 16, 16)]
                m = (sl >= lo) & (sl < lo + SW)
                plsc.store_compressed(lslot.at[pl.ds(cnt, 16)], sl, mask=m)
                plsc.store_compressed(lsrc.at[pl.ds(cnt, 16)],
                                      srcb[pl.ds(g * 16, 16)], mask=m)
                cnt = cnt + plsc.all_reduce_population_count(m)[0]
                full = cnt > LCAP - 16

                @pl.when(full)
                def _():
                    drain(cnt)
                return jnp.where(full, 0, cnt)
            return lax.fori_loop(0, CH // 16, grp, cnt)
        return lax.fori_loop(0, nch, chunk, cnt)

    cnt = lax.fori_loop(0, NW, one_list, jnp.int32(0))
    drain(cnt)
    pltpu.sync_copy(acc, agg_hbm.at[pl.ds(lo * DP, SW * DP)])


def _merge(cslot, csrc, cnts, y):
    return pl.kernel(
        _merge_body,
        out_type=jax.ShapeDtypeStruct((U * DP,), jnp.float32),
        mesh=_MESH,
        compiler_params=_SC_PARAMS,
        scratch_types=[pltpu.VMEM((NW * 16,), jnp.int32),
                       pltpu.VMEM((CH,), jnp.int32),
                       pltpu.VMEM((CH,), jnp.int32),
                       pltpu.VMEM((LBUF,), jnp.int32),
                       pltpu.VMEM((LBUF,), jnp.int32),
                       pltpu.VMEM((16, DP), jnp.float32),
                       pltpu.VMEM((SW * DP,), jnp.float32),
                       pltpu.SemaphoreType.DMA],
    )(cslot, csrc, cnts, y)


def _tgt_body(tgt_hbm, dmap_hbm, agg_hbm, out_hbm, tgt_v, dmap_v, rows, sem):
    nt = U // NW
    base = _wid() * nt
    pltpu.sync_copy(tgt_hbm.at[pl.ds(base, nt)], tgt_v)
    pltpu.sync_copy(dmap_hbm, dmap_v)
    for q in range(nt // 16):
        t = tgt_v[pl.ds(q * 16, 16)]
        sl = plsc.load_gather(dmap_v, [t])
        pltpu.async_copy(agg_hbm.at[sl], rows, sem).wait()
        pltpu.sync_copy(rows, out_hbm.at[pl.ds(base + q * 16, 16)])


def _gather_targets(tgt, dmap, agg2):
    return pl.kernel(
        _tgt_body,
        out_type=jax.ShapeDtypeStruct((U, DP), jnp.float32),
        mesh=_MESH,
        compiler_params=_SC_PARAMS,
        scratch_types=[pltpu.VMEM((U // NW,), jnp.int32),
                       pltpu.VMEM((NP,), jnp.int32),
                       pltpu.VMEM((16, DP), jnp.float32),
                       pltpu.SemaphoreType.DMA],
    )(tgt, dmap, agg2)


def _heads_body(rt_ref, rm_ref, wo_ref, bo_ref,
                wt1_ref, bt1_ref, wt2_ref, bt2_ref,
                wm1_ref, bm1_ref, wm2_ref, bm2_ref,
                ot_ref, om_ref):
    wo = wo_ref[...]
    bo = bo_ref[...]

    def group(a_ref, w1_ref, b1_ref, w2_ref, b2_ref, o_ref):
        a = a_ref[...][:, :D]
        a = jnp.where(jnp.isfinite(a), a, 0.0)
        r = jnp.dot(a, wo, preferred_element_type=jnp.float32) + bo
        for k in range(K):
            h = jnp.maximum(
                jnp.dot(r, w1_ref[k], preferred_element_type=jnp.float32)
                + b1_ref[...][k][None, :], 0.0)
            o_ref[k] = (jnp.dot(h, w2_ref[k], preferred_element_type=jnp.float32)
                        + b2_ref[...][k][None, :])

    group(rt_ref, wt1_ref, bt1_ref, wt2_ref, bt2_ref, ot_ref)
    group(rm_ref, wm1_ref, bm1_ref, wm2_ref, bm2_ref, om_ref)


def _heads(rt_agg, rm_agg, W_out, b_out2, Wt1, bt1, Wt2p, bt2p,
           Wm1, bm1, Wm2p, bm2p):
    return pl.pallas_call(
        _heads_body,
        out_shape=[jax.ShapeDtypeStruct((K, T, OP), jnp.float32),
                   jax.ShapeDtypeStruct((K, T, OP), jnp.float32)],
    )(rt_agg, rm_agg, W_out, b_out2, Wt1, bt1, Wt2p, bt2p,
      Wm1, bm1, Wm2p, bm2p)


def kernel(x, edge_index, target_tasklets, target_map_entry,
           W_msg, W_out, b_out, Wt1, bt1, Wt2, bt2, Wm1, bm1, Wm2, bm2):
    f32 = jnp.float32
    w_p = jnp.zeros((D, DP), f32).at[:, :D].set(W_msg)
    y = _msg_matmul(x, w_p)

    tgt = jnp.concatenate([target_tasklets[0], target_map_entry[0]])
    dmap = _build_dmap(tgt)

    cslot, csrc, cnts = _prefilter(edge_index[1], edge_index[0], dmap)
    agg_flat = _merge(cslot, csrc, cnts, y)
    agg2 = agg_flat.reshape(U, DP)

    rtm = _gather_targets(tgt, dmap, agg2)
    rt_agg, rm_agg = rtm[:T], rtm[T:]

    b_out2 = b_out.reshape(1, D)
    Wt2p = jnp.zeros((K, D, OP), f32).at[:, :, :2].set(Wt2)
    bt2p = jnp.zeros((K, OP), f32).at[:, :2].set(bt2)
    Wm2p = jnp.zeros((K, D, OP), f32).at[:, :, :2].set(Wm2)
    bm2p = jnp.zeros((K, OP), f32).at[:, :2].set(bm2)
    ot, om = _heads(rt_agg, rm_agg, W_out, b_out2, Wt1, bt1, Wt2p, bt2p,
                    Wm1, bm1, Wm2p, bm2p)
    return ot[:, :, :2], om[:, :, :2]


# speedup vs baseline: 12.1565x; 1.2652x over previous
import jax
import jax.numpy as jnp
from jax import lax
from jax.experimental import pallas as pl
from jax.experimental.pallas import tpu as pltpu
from jax.experimental.pallas import tpu_sc as plsc

N = 50000
E = 800000
D = 70
K = 4
T = 2048

NC, NS = 2, 16
NW = NC * NS
DP = 128
SN = 1568
NP = NW * SN
U = 2 * T
CH = 2048

EW1 = 25024
NFULL0, TAIL0 = 12, 448
NFULL1, TAIL1 = 11, 1728
CROW = 13 * CH
CBUF = CROW + 16

NSG, NQ = 8, 4
SW2 = U // NSG
LPQ = NW // NQ

LCAP = 2048
KDR = 64
LBUF = LCAP + KDR + 16
OP = 128

_MESH = plsc.VectorSubcoreMesh(core_axis_name="c", subcore_axis_name="s",
                               num_cores=NC, num_subcores=NS)
_SC_PARAMS = pltpu.CompilerParams(needs_layout_passes=False)


def _wid():
    return lax.axis_index("s") * NC + lax.axis_index("c")


def _mm_body(x_ref, w_ref, o_ref):
    o_ref[...] = jnp.dot(x_ref[...], w_ref[...],
                         preferred_element_type=jnp.float32)


def _msg_matmul(x, w_p):
    BM = 512
    return pl.pallas_call(
        _mm_body,
        grid=(pl.cdiv(N, BM),),
        in_specs=[pl.BlockSpec((BM, D), lambda i: (i, 0)),
                  pl.BlockSpec((D, DP), lambda i: (0, 0))],
        out_specs=pl.BlockSpec((BM, DP), lambda i: (i, 0)),
        out_shape=jax.ShapeDtypeStruct((N, DP), jnp.float32),
    )(x, w_p)


def _dmap_body(tgt_hbm, dmap_hbm, tgt_v, loc_v):
    lo = _wid() * SN
    neg1 = jnp.full((16,), -1, jnp.int32)

    def init(i, _):
        loc_v[pl.ds(i * 16, 16)] = neg1
        return 0
    lax.fori_loop(0, SN // 16, init, 0)

    pltpu.sync_copy(tgt_hbm, tgt_v)
    lane = lax.iota(jnp.int32, 16)

    def scat(g, _):
        t = tgt_v[pl.ds(g * 16, 16)]
        m = (t >= lo) & (t < lo + SN)
        idx = jnp.where(m, t - lo, 0)
        plsc.store_scatter(loc_v, [idx], g * 16 + lane, mask=m)
        return 0
    lax.fori_loop(0, U // 16, scat, 0)

    pltpu.sync_copy(loc_v, dmap_hbm.at[pl.ds(lo, SN)])


def _build_dmap(tgt):
    return pl.kernel(
        _dmap_body,
        out_type=jax.ShapeDtypeStruct((NP,), jnp.int32),
        mesh=_MESH,
        compiler_params=_SC_PARAMS,
        scratch_types=[pltpu.VMEM((U,), jnp.int32),
                       pltpu.VMEM((SN,), jnp.int32)],
    )(tgt)


def _pre_body(dst_hbm, src_hbm, dmap_hbm, cpk_hbm, cnts_hbm,
              dmap_v, dstb, srcb, lpk, cstage):
    w = _wid()
    pltpu.sync_copy(dmap_hbm, dmap_v)
    ebase = w * EW1

    def scan_groups(ngrp, cnt):
        def grp(g, cnt):
            dv = dstb[pl.ds(g * 16, 16)]
            sl = plsc.load_gather(dmap_v, [dv])
            m = sl >= 0
            pk = (sl << 16) | srcb[pl.ds(g * 16, 16)]
            plsc.store_compressed(lpk.at[pl.ds(cnt, 16)], pk, mask=m)
            return cnt + plsc.all_reduce_population_count(m)[0]
        return lax.fori_loop(0, ngrp, grp, cnt)

    nfull = jnp.where(w < NW - 1, NFULL0, NFULL1)

    def chunk(ci, cnt):
        pltpu.sync_copy(dst_hbm.at[pl.ds(ebase + ci * CH, CH)], dstb)
        pltpu.sync_copy(src_hbm.at[pl.ds(ebase + ci * CH, CH)], srcb)
        return scan_groups(CH // 16, cnt)

    cnt = lax.fori_loop(0, nfull, chunk, jnp.int32(0))

    @pl.when(w < NW - 1)
    def _():
        pltpu.sync_copy(dst_hbm.at[pl.ds(ebase + NFULL0 * CH, TAIL0)],
                        dstb.at[pl.ds(0, TAIL0)])
        pltpu.sync_copy(src_hbm.at[pl.ds(ebase + NFULL0 * CH, TAIL0)],
                        srcb.at[pl.ds(0, TAIL0)])

    @pl.when(w == NW - 1)
    def _():
        pltpu.sync_copy(dst_hbm.at[pl.ds(ebase + NFULL1 * CH, TAIL1)],
                        dstb.at[pl.ds(0, TAIL1)])
        pltpu.sync_copy(src_hbm.at[pl.ds(ebase + NFULL1 * CH, TAIL1)],
                        srcb.at[pl.ds(0, TAIL1)])

    ntailg = jnp.where(w < NW - 1, TAIL0 // 16, TAIL1 // 16)
    cnt = scan_groups(ntailg, cnt)

    roundup = ((cnt + CH - 1) // CH) * CH
    neg1 = jnp.full((16,), -1, jnp.int32)

    def fill(j, _):
        lpk[pl.ds(cnt + j * 16, 16)] = neg1
        return 0
    lax.fori_loop(0, (roundup - cnt + 15) // 16, fill, 0)

    pltpu.sync_copy(lpk.at[pl.ds(0, CROW)], cpk_hbm.at[pl.ds(w * CROW, CROW)])
    cstage[pl.ds(0, 16)] = jnp.zeros((16,), jnp.int32) + cnt
    pltpu.sync_copy(cstage, cnts_hbm.at[pl.ds(w * 16, 16)])


def _prefilter(dst, src, dmap):
    return pl.kernel(
        _pre_body,
        out_type=[jax.ShapeDtypeStruct((NW * CROW,), jnp.int32),
                  jax.ShapeDtypeStruct((NW * 16,), jnp.int32)],
        mesh=_MESH,
        compiler_params=_SC_PARAMS,
        scratch_types=[pltpu.VMEM((NP,), jnp.int32),
                       pltpu.VMEM((CH,), jnp.int32),
                       pltpu.VMEM((CH,), jnp.int32),
                       pltpu.VMEM((CBUF,), jnp.int32),
                       pltpu.VMEM((16,), jnp.int32)],
    )(dst, src, dmap)


def _merge_body(cpk_hbm, cnts_hbm, y_hbm, aggp_hbm,
                cntsv, pkb, lpk, stage_a, stage_b, rows_a, rows_b,
                acc, sem_a, sem_b):
    w = _wid()
    sgrp = w // NQ
    quarter = w % NQ
    lo = sgrp * SW2
    pltpu.sync_copy(cnts_hbm, cntsv)

    ninf = jnp.full((16,), -jnp.inf, jnp.float32)

    def init_acc(i, _):
        acc[pl.ds(i * 16, 16)] = ninf
        return 0
    lax.fori_loop(0, (SW2 + 1) * DP // 16, init_acc, 0)

    dummy = jnp.zeros((16,), jnp.int32) + ((lo + SW2) << 16)

    def issue(sgi, stage, buf, sem):
        for k in range(KDR // 16):
            pk = lpk[pl.ds(sgi * KDR + k * 16, 16)]
            stage[pl.ds(k * 16, 16)] = pk & 0xFFFF
        return pltpu.async_copy(y_hbm.at[stage], buf, sem)

    def accum(sgi, buf):
        for k in range(KDR // 16):
            pk = lpk[pl.ds(sgi * KDR + k * 16, 16)]
            for l in range(16):
                b = ((pk[l] >> 16) - lo) * DP
                for c in range(DP // 16):
                    av = acc[pl.ds(b + c * 16, 16)]
                    rv = buf[k * 16 + l, pl.ds(c * 16, 16)]
                    acc[pl.ds(b + c * 16, 16)] = jnp.maximum(av, rv)

    def wait_for(stage, buf, sem):
        pltpu.make_async_copy(y_hbm.at[stage], buf, sem).wait()

    def drain(n):
        roundup = ((n + KDR - 1) // KDR) * KDR

        def fill(j, _):
            lpk[pl.ds(n + j * 16, 16)] = dummy
            return 0
        lax.fori_loop(0, (roundup - n + 15) // 16, fill, 0)
        nsg = roundup // KDR

        @pl.when(nsg > 0)
        def _():
            issue(0, stage_a, rows_a, sem_a)

            def pair(p, _):
                @pl.when(2 * p + 1 < nsg)
                def _():
                    issue(2 * p + 1, stage_b, rows_b, sem_b)
                wait_for(stage_a, rows_a, sem_a)
                accum(2 * p, rows_a)

                @pl.when(2 * p + 2 < nsg)
                def _():
                    issue(2 * p + 2, stage_a, rows_a, sem_a)

                @pl.when(2 * p + 1 < nsg)
                def _():
                    wait_for(stage_b, rows_b, sem_b)
                    accum(2 * p + 1, rows_b)
                return 0
            lax.fori_loop(0, (nsg + 1) // 2, pair, 0)

    def one_list(j, cnt):
        v = quarter * LPQ + j
        nch = (cntsv[pl.ds(v * 16, 16)][0] + CH - 1) // CH

        def chunk(ci, cnt):
            pltpu.sync_copy(cpk_hbm.at[pl.ds(v * CROW + ci * CH, CH)], pkb)

            def grp(g, cnt):
                pk = pkb[pl.ds(g * 16, 16)]
                sl = pk >> 16
                m = (sl >= lo) & (sl < lo + SW2)
                plsc.store_compressed(lpk.at[pl.ds(cnt, 16)], pk, mask=m)
                cnt = cnt + plsc.all_reduce_population_count(m)[0]
                full = cnt > LCAP - 16

                @pl.when(full)
                def _():
                    drain(cnt)
                return jnp.where(full, 0, cnt)
            return lax.fori_loop(0, CH // 16, grp, cnt)
        return lax.fori_loop(0, nch, chunk, cnt)

    cnt = lax.fori_loop(0, LPQ, one_list, jnp.int32(0))
    drain(cnt)
    pltpu.sync_copy(acc.at[pl.ds(0, SW2 * DP)],
                    aggp_hbm.at[pl.ds((quarter * U + lo) * DP, SW2 * DP)])


def _merge(cpk, cnts, y):
    return pl.kernel(
        _merge_body,
        out_type=jax.ShapeDtypeStruct((NQ * U * DP,), jnp.float32),
        mesh=_MESH,
        compiler_params=_SC_PARAMS,
        scratch_types=[pltpu.VMEM((NW * 16,), jnp.int32),
                       pltpu.VMEM((CH,), jnp.int32),
                       pltpu.VMEM((LBUF,), jnp.int32),
                       pltpu.VMEM((KDR,), jnp.int32),
                       pltpu.VMEM((KDR,), jnp.int32),
                       pltpu.VMEM((KDR, DP), jnp.float32),
                       pltpu.VMEM((KDR, DP), jnp.float32),
                       pltpu.VMEM(((SW2 + 1) * DP,), jnp.float32),
                       pltpu.SemaphoreType.DMA,
                       pltpu.SemaphoreType.DMA],
    )(cpk, cnts, y)


def _tgt_body(tgt_hbm, dmap_hbm, aggp_hbm, out_hbm,
              tgt_v, dmap_v, r0, r1, r2, r3, rout, sem):
    nt = U // NW
    base = _wid() * nt
    pltpu.sync_copy(tgt_hbm.at[pl.ds(base, nt)], tgt_v)
    pltpu.sync_copy(dmap_hbm, dmap_v)
    bufs = [r0, r1, r2, r3]
    for q in range(nt // 16):
        t = tgt_v[pl.ds(q * 16, 16)]
        sl = plsc.load_gather(dmap_v, [t])
        descs = [pltpu.async_copy(aggp_hbm.at[sl + p * U], bufs[p], sem)
                 for p in range(NQ)]
        for d in descs:
            d.wait()
        for r in range(16):
            for c in range(DP // 16):
                m01 = jnp.maximum(r0[r, pl.ds(c * 16, 16)],
                                  r1[r, pl.ds(c * 16, 16)])
                m23 = jnp.maximum(r2[r, pl.ds(c * 16, 16)],
                                  r3[r, pl.ds(c * 16, 16)])
                rout[r, pl.ds(c * 16, 16)] = jnp.maximum(m01, m23)
        pltpu.sync_copy(rout, out_hbm.at[pl.ds(base + q * 16, 16)])


def _gather_targets(tgt, dmap, aggp2):
    return pl.kernel(
        _tgt_body,
        out_type=jax.ShapeDtypeStruct((U, DP), jnp.float32),
        mesh=_MESH,
        compiler_params=_SC_PARAMS,
        scratch_types=[pltpu.VMEM((U // NW,), jnp.int32),
                       pltpu.VMEM((NP,), jnp.int32),
                       pltpu.VMEM((16, DP), jnp.float32),
                       pltpu.VMEM((16, DP), jnp.float32),
                       pltpu.VMEM((16, DP), jnp.float32),
                       pltpu.VMEM((16, DP), jnp.float32),
                       pltpu.VMEM((16, DP), jnp.float32),
                       pltpu.SemaphoreType.DMA],
    )(tgt, dmap, aggp2)


def _heads_body(rt_ref, rm_ref, wo_ref, bo_ref,
                wt1_ref, bt1_ref, wt2_ref, bt2_ref,
                wm1_ref, bm1_ref, wm2_ref, bm2_ref,
                ot_ref, om_ref):
    wo = wo_ref[...]
    bo = bo_ref[...]

    def group(a_ref, w1_ref, b1_ref, w2_ref, b2_ref, o_ref):
        a = a_ref[...][:, :D]
        a = jnp.where(jnp.isfinite(a), a, 0.0)
        r = jnp.dot(a, wo, preferred_element_type=jnp.float32) + bo
        for k in range(K):
            h = jnp.maximum(
                jnp.dot(r, w1_ref[k], preferred_element_type=jnp.float32)
                + b1_ref[...][k][None, :], 0.0)
            o_ref[k] = (jnp.dot(h, w2_ref[k], preferred_element_type=jnp.float32)
                        + b2_ref[...][k][None, :])

    group(rt_ref, wt1_ref, bt1_ref, wt2_ref, bt2_ref, ot_ref)
    group(rm_ref, wm1_ref, bm1_ref, wm2_ref, bm2_ref, om_ref)


def _heads(rt_agg, rm_agg, W_out, b_out2, Wt1, bt1, Wt2p, bt2p,
           Wm1, bm1, Wm2p, bm2p):
    return pl.pallas_call(
        _heads_body,
        out_shape=[jax.ShapeDtypeStruct((K, T, OP), jnp.float32),
                   jax.ShapeDtypeStruct((K, T, OP), jnp.float32)],
    )(rt_agg, rm_agg, W_out, b_out2, Wt1, bt1, Wt2p, bt2p,
      Wm1, bm1, Wm2p, bm2p)


def kernel(x, edge_index, target_tasklets, target_map_entry,
           W_msg, W_out, b_out, Wt1, bt1, Wt2, bt2, Wm1, bm1, Wm2, bm2):
    f32 = jnp.float32
    w_p = jnp.zeros((D, DP), f32).at[:, :D].set(W_msg)
    y = _msg_matmul(x, w_p)

    tgt = jnp.concatenate([target_tasklets[0], target_map_entry[0]])
    dmap = _build_dmap(tgt)

    cpk, cnts = _prefilter(edge_index[1], edge_index[0], dmap)
    aggp_flat = _merge(cpk, cnts, y)
    aggp2 = aggp_flat.reshape(NQ * U, DP)

    rtm = _gather_targets(tgt, dmap, aggp2)
    rt_agg, rm_agg = rtm[:T], rtm[T:]

    b_out2 = b_out.reshape(1, D)
    Wt2p = jnp.zeros((K, D, OP), f32).at[:, :, :2].set(Wt2)
    bt2p = jnp.zeros((K, OP), f32).at[:, :2].set(bt2)
    Wm2p = jnp.zeros((K, D, OP), f32).at[:, :, :2].set(Wm2)
    bm2p = jnp.zeros((K, OP), f32).at[:, :2].set(bm2)
    ot, om = _heads(rt_agg, rm_agg, W_out, b_out2, Wt1, bt1, Wt2p, bt2p,
                    Wm1, bm1, Wm2p, bm2p)
    return ot[:, :, :2], om[:, :, :2]


# speedup vs baseline: 12.7286x; 1.0471x over previous
import jax
import jax.numpy as jnp
from jax import lax
from jax.experimental import pallas as pl
from jax.experimental.pallas import tpu as pltpu
from jax.experimental.pallas import tpu_sc as plsc

N = 50000
E = 800000
D = 70
K = 4
T = 2048

NC, NS = 2, 16
NW = NC * NS
DP = 128
AW = 80
SN = 1568
NP = NW * SN
U = 2 * T
CH = 2048

EW1 = 25024
NFULL0, TAIL0 = 12, 448
NFULL1, TAIL1 = 11, 1728
CROW = 7 * CH
CBUF = CROW + 16
NL = 2 * NW

NSG, NQ = 4, 8
SW2 = U // NSG
LPQ = NL // NQ

LCAP = 2048
KDR = 64
LBUF = LCAP + KDR + 16
ACCW = (SW2 + 1) * AW
ACCA = ((ACCW // 16 + 7) // 8) * 128 + 16

_MESH = plsc.VectorSubcoreMesh(core_axis_name="c", subcore_axis_name="s",
                               num_cores=NC, num_subcores=NS)
_SC_PARAMS = pltpu.CompilerParams(needs_layout_passes=False)


def _wid():
    return lax.axis_index("s") * NC + lax.axis_index("c")


def _mm_body(x_ref, w_ref, o_ref):
    o_ref[...] = jnp.dot(x_ref[...], w_ref[...],
                         preferred_element_type=jnp.float32)


def _msg_matmul(x, w_p):
    BM = 512
    return pl.pallas_call(
        _mm_body,
        grid=(pl.cdiv(N, BM),),
        in_specs=[pl.BlockSpec((BM, D), lambda i: (i, 0)),
                  pl.BlockSpec((D, DP), lambda i: (0, 0))],
        out_specs=pl.BlockSpec((BM, DP), lambda i: (i, 0)),
        out_shape=jax.ShapeDtypeStruct((N, DP), jnp.float32),
    )(x, w_p)


def _dmap_body(tgt_hbm, dmap_hbm, tgt_v, loc_v):
    lo = _wid() * SN
    neg1 = jnp.full((16,), -1, jnp.int32)

    def init(i, _):
        loc_v[pl.ds(i * 16, 16)] = neg1
        return 0
    lax.fori_loop(0, SN // 16, init, 0)

    pltpu.sync_copy(tgt_hbm, tgt_v)
    lane = lax.iota(jnp.int32, 16)

    def scat(g, _):
        t = tgt_v[pl.ds(g * 16, 16)]
        m = (t >= lo) & (t < lo + SN)
        idx = jnp.where(m, t - lo, 0)
        plsc.store_scatter(loc_v, [idx], g * 16 + lane, mask=m)
        return 0
    lax.fori_loop(0, U // 16, scat, 0)

    pltpu.sync_copy(loc_v, dmap_hbm.at[pl.ds(lo, SN)])


def _build_dmap(tgt):
    return pl.kernel(
        _dmap_body,
        out_type=jax.ShapeDtypeStruct((NP,), jnp.int32),
        mesh=_MESH,
        compiler_params=_SC_PARAMS,
        scratch_types=[pltpu.VMEM((U,), jnp.int32),
                       pltpu.VMEM((SN,), jnp.int32)],
    )(tgt)


def _pre_body(dst_hbm, src_hbm, dmap_hbm, cpk_hbm, cnts_hbm,
              dmap_v, dstb, srcb, lpa, lpb, cstage):
    w = _wid()
    pltpu.sync_copy(dmap_hbm, dmap_v)
    ebase = w * EW1

    def scan_pairs(npair, carry):
        cnta, cntb = carry

        def pair(g, carry):
            cnta, cntb = carry
            dva = dstb[pl.ds(g * 32, 16)]
            dvb = dstb[pl.ds(g * 32 + 16, 16)]
            sla = plsc.load_gather(dmap_v, [dva])
            slb = plsc.load_gather(dmap_v, [dvb])
            ma = sla >= 0
            mb = slb >= 0
            pka = (sla << 16) | srcb[pl.ds(g * 32, 16)]
            pkb2 = (slb << 16) | srcb[pl.ds(g * 32 + 16, 16)]
            plsc.store_compressed(lpa.at[pl.ds(cnta, 16)], pka, mask=ma)
            plsc.store_compressed(lpb.at[pl.ds(cntb, 16)], pkb2, mask=mb)
            return (cnta + plsc.all_reduce_population_count(ma)[0],
                    cntb + plsc.all_reduce_population_count(mb)[0])
        return lax.fori_loop(0, npair, pair, (cnta, cntb))

    nfull = jnp.where(w < NW - 1, NFULL0, NFULL1)

    def chunk(ci, carry):
        pltpu.sync_copy(dst_hbm.at[pl.ds(ebase + ci * CH, CH)], dstb)
        pltpu.sync_copy(src_hbm.at[pl.ds(ebase + ci * CH, CH)], srcb)
        return scan_pairs(CH // 32, carry)

    carry = lax.fori_loop(0, nfull, chunk, (jnp.int32(0), jnp.int32(0)))

    @pl.when(w < NW - 1)
    def _():
        pltpu.sync_copy(dst_hbm.at[pl.ds(ebase + NFULL0 * CH, TAIL0)],
                        dstb.at[pl.ds(0, TAIL0)])
        pltpu.sync_copy(src_hbm.at[pl.ds(ebase + NFULL0 * CH, TAIL0)],
                        srcb.at[pl.ds(0, TAIL0)])

    @pl.when(w == NW - 1)
    def _():
        pltpu.sync_copy(dst_hbm.at[pl.ds(ebase + NFULL1 * CH, TAIL1)],
                        dstb.at[pl.ds(0, TAIL1)])
        pltpu.sync_copy(src_hbm.at[pl.ds(ebase + NFULL1 * CH, TAIL1)],
                        srcb.at[pl.ds(0, TAIL1)])

    ntailp = jnp.where(w < NW - 1, TAIL0 // 32, TAIL1 // 32)
    cnta, cntb = scan_pairs(ntailp, carry)

    neg1 = jnp.full((16,), -1, jnp.int32)

    def flush(lp, cnt, li):
        roundup = ((cnt + CH - 1) // CH) * CH

        def fill(j, _):
            lp[pl.ds(cnt + j * 16, 16)] = neg1
            return 0
        lax.fori_loop(0, (roundup - cnt + 15) // 16, fill, 0)
        pltpu.sync_copy(lp.at[pl.ds(0, CROW)],
                        cpk_hbm.at[pl.ds(li * CROW, CROW)])
        cstage[pl.ds(0, 16)] = jnp.zeros((16,), jnp.int32) + cnt
        pltpu.sync_copy(cstage, cnts_hbm.at[pl.ds(li * 16, 16)])

    flush(lpa, cnta, 2 * w)
    flush(lpb, cntb, 2 * w + 1)


def _prefilter(dst, src, dmap):
    return pl.kernel(
        _pre_body,
        out_type=[jax.ShapeDtypeStruct((NL * CROW,), jnp.int32),
                  jax.ShapeDtypeStruct((NL * 16,), jnp.int32)],
        mesh=_MESH,
        compiler_params=_SC_PARAMS,
        scratch_types=[pltpu.VMEM((NP,), jnp.int32),
                       pltpu.VMEM((CH,), jnp.int32),
                       pltpu.VMEM((CH,), jnp.int32),
                       pltpu.VMEM((CBUF,), jnp.int32),
                       pltpu.VMEM((CBUF,), jnp.int32),
                       pltpu.VMEM((16,), jnp.int32)],
    )(dst, src, dmap)


def _merge_body(cpk_hbm, cnts_hbm, y_hbm, aggp_hbm,
                cntsv, pkb, lpk, stage_a, stage_b, rows_a, rows_b,
                acc, sem_a, sem_b):
    w = _wid()
    sgrp = w // NQ
    quarter = w % NQ
    lo = sgrp * SW2
    pltpu.sync_copy(cnts_hbm, cntsv)

    ninf = jnp.full((16,), -jnp.inf, jnp.float32)

    def init_acc(i, _):
        for j in range(8):
            acc[pl.ds(i * 128 + j * 16, 16)] = ninf
        return 0
    lax.fori_loop(0, (ACCW // 16 + 7) // 8, init_acc, 0)

    dummy = jnp.zeros((16,), jnp.int32) + ((lo + SW2) << 16)

    def issue(sgi, stage, buf, sem):
        for k in range(KDR // 16):
            pk = lpk[pl.ds(sgi * KDR + k * 16, 16)]
            stage[pl.ds(k * 16, 16)] = pk & 0xFFFF
        return pltpu.async_copy(y_hbm.at[stage], buf, sem)

    def accum(sgi, buf):
        for k in range(KDR // 16):
            pk = lpk[pl.ds(sgi * KDR + k * 16, 16)]
            for l in range(16):
                b = ((pk[l] >> 16) - lo) * AW
                for c in range(AW // 16):
                    av = acc[pl.ds(b + c * 16, 16)]
                    rv = buf[k * 16 + l, pl.ds(c * 16, 16)]
                    acc[pl.ds(b + c * 16, 16)] = jnp.maximum(av, rv)

    def wait_for(stage, buf, sem):
        pltpu.make_async_copy(y_hbm.at[stage], buf, sem).wait()

    def drain(n):
        roundup = ((n + KDR - 1) // KDR) * KDR

        def fill(j, _):
            lpk[pl.ds(n + j * 16, 16)] = dummy
            return 0
        lax.fori_loop(0, (roundup - n + 15) // 16, fill, 0)
        nsg = roundup // KDR

        @pl.when(nsg > 0)
        def _():
            issue(0, stage_a, rows_a, sem_a)

            def pair(p, _):
                @pl.when(2 * p + 1 < nsg)
                def _():
                    issue(2 * p + 1, stage_b, rows_b, sem_b)
                wait_for(stage_a, rows_a, sem_a)
                accum(2 * p, rows_a)

                @pl.when(2 * p + 2 < nsg)
                def _():
                    issue(2 * p + 2, stage_a, rows_a, sem_a)

                @pl.when(2 * p + 1 < nsg)
                def _():
                    wait_for(stage_b, rows_b, sem_b)
                    accum(2 * p + 1, rows_b)
                return 0
            lax.fori_loop(0, (nsg + 1) // 2, pair, 0)

    def one_list(j, cnt):
        v = quarter * LPQ + j
        nch = (cntsv[pl.ds(v * 16, 16)][0] + CH - 1) // CH

        def chunk(ci, cnt):
            pltpu.sync_copy(cpk_hbm.at[pl.ds(v * CROW + ci * CH, CH)], pkb)

            def grp(g, cnt):
                pk = pkb[pl.ds(g * 16, 16)]
                sl = pk >> 16
                m = (sl >= lo) & (sl < lo + SW2)
                plsc.store_compressed(lpk.at[pl.ds(cnt, 16)], pk, mask=m)
                cnt = cnt + plsc.all_reduce_population_count(m)[0]
                full = cnt > LCAP - 16

                @pl.when(full)
                def _():
                    drain(cnt)
                return jnp.where(full, 0, cnt)
            return lax.fori_loop(0, CH // 16, grp, cnt)
        return lax.fori_loop(0, nch, chunk, cnt)

    cnt = lax.fori_loop(0, LPQ, one_list, jnp.int32(0))
    drain(cnt)
    pltpu.sync_copy(acc.at[pl.ds(0, SW2 * AW)],
                    aggp_hbm.at[pl.ds((quarter * U + lo) * AW, SW2 * AW)])


def _merge(cpk, cnts, y):
    return pl.kernel(
        _merge_body,
        out_type=jax.ShapeDtypeStruct((NQ * U * AW,), jnp.float32),
        mesh=_MESH,
        compiler_params=_SC_PARAMS,
        scratch_types=[pltpu.VMEM((NL * 16,), jnp.int32),
                       pltpu.VMEM((CH,), jnp.int32),
                       pltpu.VMEM((LBUF,), jnp.int32),
                       pltpu.VMEM((KDR,), jnp.int32),
                       pltpu.VMEM((KDR,), jnp.int32),
                       pltpu.VMEM((KDR, DP), jnp.float32),
                       pltpu.VMEM((KDR, DP), jnp.float32),
                       pltpu.VMEM((ACCA,), jnp.float32),
                       pltpu.SemaphoreType.DMA,
                       pltpu.SemaphoreType.DMA],
    )(cpk, cnts, y)


def _heads_body(aggp_ref, wo_ref, bo_ref,
                wt1_ref, bt1_ref, wt2_ref, bt2_ref,
                wm1_ref, bm1_ref, wm2_ref, bm2_ref,
                ot_ref, om_ref):
    a = jnp.max(aggp_ref[...], axis=0)[:, :D]
    a = jnp.where(jnp.isfinite(a), a, 0.0)
    r = jnp.dot(a, wo_ref[...], preferred_element_type=jnp.float32) + bo_ref[...]

    def group(w1_ref, b1_ref, w2_ref, b2_ref, o_ref):
        for k in range(K):
            h = jnp.maximum(
                jnp.dot(r, w1_ref[k], preferred_element_type=jnp.float32)
                + b1_ref[...][k][None, :], 0.0)
            o_ref[k] = (jnp.dot(h, w2_ref[k], preferred_element_type=jnp.float32)
                        + b2_ref[...][k][None, :])

    group(wt1_ref, bt1_ref, wt2_ref, bt2_ref, ot_ref)
    group(wm1_ref, bm1_ref, wm2_ref, bm2_ref, om_ref)


def _heads(aggp3, W_out, b_out2, Wt1, bt1, Wt2p, bt2p, Wm1, bm1, Wm2p, bm2p):
    return pl.pallas_call(
        _heads_body,
        out_shape=[jax.ShapeDtypeStruct((K, U, 4), jnp.float32),
                   jax.ShapeDtypeStruct((K, U, 4), jnp.float32)],
    )(aggp3, W_out, b_out2, Wt1, bt1, Wt2p, bt2p, Wm1, bm1, Wm2p, bm2p)


def _out_body(ht_hbm, hm_hbm, tgt_hbm, dmap_hbm, rt_hbm, rm_hbm,
              htv, hmv, tgtv, dmap_v, ob):
    w = _wid()
    base = w * (K * T * 2 // NW)
    myk = base // (T * 2)
    pltpu.sync_copy(ht_hbm.at[pl.ds(myk * U * 4, U * 4)], htv)
    pltpu.sync_copy(hm_hbm.at[pl.ds(myk * U * 4, U * 4)], hmv)
    pltpu.sync_copy(tgt_hbm, tgtv)
    pltpu.sync_copy(dmap_hbm, dmap_v)
    lane = lax.iota(jnp.int32, 16)

    def emit(src_v, tgt_off, out_hbm):
        def g_body(g, _):
            p = base + g * 16 + lane
            rr = (p >> 1) & (T - 1)
            j = p & 1
            t = plsc.load_gather(tgtv, [rr + tgt_off])
            sl = plsc.load_gather(dmap_v, [t])
            val = plsc.load_gather(src_v, [sl * 4 + j])
            ob[pl.ds(g * 16, 16)] = val
            return 0
        lax.fori_loop(0, (K * T * 2 // NW) // 16, g_body, 0)
        pltpu.sync_copy(ob, out_hbm.at[pl.ds(base, K * T * 2 // NW)])

    emit(htv, 0, rt_hbm)
    emit(hmv, T, rm_hbm)


def _out_gather(ht, hm, tgt, dmap):
    return pl.kernel(
        _out_body,
        out_type=[jax.ShapeDtypeStruct((K * T * 2,), jnp.float32),
                  jax.ShapeDtypeStruct((K * T * 2,), jnp.float32)],
        mesh=_MESH,
        compiler_params=_SC_PARAMS,
        scratch_types=[pltpu.VMEM((U * 4,), jnp.float32),
                       pltpu.VMEM((U * 4,), jnp.float32),
                       pltpu.VMEM((U,), jnp.int32),
                       pltpu.VMEM((NP,), jnp.int32),
                       pltpu.VMEM((K * T * 2 // NW,), jnp.float32)],
    )(ht, hm, tgt, dmap)


def kernel(x, edge_index, target_tasklets, target_map_entry,
           W_msg, W_out, b_out, Wt1, bt1, Wt2, bt2, Wm1, bm1, Wm2, bm2):
    f32 = jnp.float32
    w_p = jnp.zeros((D, DP), f32).at[:, :D].set(W_msg)
    y = _msg_matmul(x, w_p)

    tgt = jnp.concatenate([target_tasklets[0], target_map_entry[0]])
    dmap = _build_dmap(tgt)

    cpk, cnts = _prefilter(edge_index[1], edge_index[0], dmap)
    aggp_flat = _merge(cpk, cnts, y)
    aggp3 = aggp_flat.reshape(NQ, U, AW)

    b_out2 = b_out.reshape(1, D)
    Wt2p = jnp.zeros((K, D, 4), f32).at[:, :, :2].set(Wt2)
    bt2p = jnp.zeros((K, 4), f32).at[:, :2].set(bt2)
    Wm2p = jnp.zeros((K, D, 4), f32).at[:, :, :2].set(Wm2)
    bm2p = jnp.zeros((K, 4), f32).at[:, :2].set(bm2)
    ht, hm = _heads(aggp3, W_out, b_out2, Wt1, bt1, Wt2p, bt2p,
                    Wm1, bm1, Wm2p, bm2p)

    rt_flat, rm_flat = _out_gather(ht.reshape(K * U * 4), hm.reshape(K * U * 4),
                                   tgt, dmap)
    return rt_flat.reshape(K, T, 2), rm_flat.reshape(K, T, 2)


# speedup vs baseline: 12.9858x; 1.0202x over previous
import jax
import jax.numpy as jnp
from jax import lax
from jax.experimental import pallas as pl
from jax.experimental.pallas import tpu as pltpu
from jax.experimental.pallas import tpu_sc as plsc

N = 50000
E = 800000
D = 70
K = 4
T = 2048

NC, NS = 2, 16
NW = NC * NS
DP = 128
AW = 80
SN = 1568
NP = NW * SN
U = 2 * T
CH = 2048

EW1 = 25024
NFULL0, TAIL0 = 12, 448
NFULL1, TAIL1 = 11, 1728
CROW = 7 * CH
CBUF = CROW + 16
NL = 2 * NW

NSG, NQ = 4, 8
SW2 = U // NSG
LPQ = NL // NQ

LCAP = 2048
KDR = 64
LBUF = LCAP + KDR + 16
ACCW = (SW2 + 1) * AW
ACCA = ((ACCW // 16 + 7) // 8) * 128 + 16

_MESH = plsc.VectorSubcoreMesh(core_axis_name="c", subcore_axis_name="s",
                               num_cores=NC, num_subcores=NS)
_SC_PARAMS = pltpu.CompilerParams(needs_layout_passes=False)


def _wid():
    return lax.axis_index("s") * NC + lax.axis_index("c")


def _mm_body(x_ref, w_ref, o_ref):
    o_ref[...] = jnp.dot(x_ref[...], w_ref[...],
                         preferred_element_type=jnp.float32)


def _msg_matmul(x, w_p):
    BM = 512
    return pl.pallas_call(
        _mm_body,
        grid=(pl.cdiv(N, BM),),
        in_specs=[pl.BlockSpec((BM, D), lambda i: (i, 0)),
                  pl.BlockSpec((D, DP), lambda i: (0, 0))],
        out_specs=pl.BlockSpec((BM, DP), lambda i: (i, 0)),
        out_shape=jax.ShapeDtypeStruct((N, DP), jnp.float32),
    )(x, w_p)


def _local_dmap(tgt_hbm, tgtv, dmap_v):
    neg1 = jnp.full((16,), -1, jnp.int32)

    def init(i, _):
        for j in range(8):
            dmap_v[pl.ds(i * 128 + j * 16, 16)] = neg1
        return 0
    lax.fori_loop(0, NP // 128, init, 0)

    pltpu.sync_copy(tgt_hbm, tgtv)
    lane = lax.iota(jnp.int32, 16)

    def scat(g, _):
        t = tgtv[pl.ds(g * 16, 16)]
        plsc.store_scatter(dmap_v, [t], g * 16 + lane)
        return 0
    lax.fori_loop(0, U // 16, scat, 0)


def _pre_body(dst_hbm, src_hbm, tgt_hbm, cpk_hbm, cnts_hbm,
              dmap_v, tgtv, dstb, srcb, lpa, lpb, cstage):
    w = _wid()
    _local_dmap(tgt_hbm, tgtv, dmap_v)
    ebase = w * EW1

    def scan_pairs(npair, carry):
        cnta, cntb = carry

        def pair(g, carry):
            cnta, cntb = carry
            dva = dstb[pl.ds(g * 32, 16)]
            dvb = dstb[pl.ds(g * 32 + 16, 16)]
            sla = plsc.load_gather(dmap_v, [dva])
            slb = plsc.load_gather(dmap_v, [dvb])
            ma = sla >= 0
            mb = slb >= 0
            pka = (sla << 16) | srcb[pl.ds(g * 32, 16)]
            pkb2 = (slb << 16) | srcb[pl.ds(g * 32 + 16, 16)]
            plsc.store_compressed(lpa.at[pl.ds(cnta, 16)], pka, mask=ma)
            plsc.store_compressed(lpb.at[pl.ds(cntb, 16)], pkb2, mask=mb)
            return (cnta + plsc.all_reduce_population_count(ma)[0],
                    cntb + plsc.all_reduce_population_count(mb)[0])
        return lax.fori_loop(0, npair, pair, (cnta, cntb))

    nfull = jnp.where(w < NW - 1, NFULL0, NFULL1)

    def chunk(ci, carry):
        pltpu.sync_copy(dst_hbm.at[pl.ds(ebase + ci * CH, CH)], dstb)
        pltpu.sync_copy(src_hbm.at[pl.ds(ebase + ci * CH, CH)], srcb)
        return scan_pairs(CH // 32, carry)

    carry = lax.fori_loop(0, nfull, chunk, (jnp.int32(0), jnp.int32(0)))

    @pl.when(w < NW - 1)
    def _():
        pltpu.sync_copy(dst_hbm.at[pl.ds(ebase + NFULL0 * CH, TAIL0)],
                        dstb.at[pl.ds(0, TAIL0)])
        pltpu.sync_copy(src_hbm.at[pl.ds(ebase + NFULL0 * CH, TAIL0)],
                        srcb.at[pl.ds(0, TAIL0)])

    @pl.when(w == NW - 1)
    def _():
        pltpu.sync_copy(dst_hbm.at[pl.ds(ebase + NFULL1 * CH, TAIL1)],
                        dstb.at[pl.ds(0, TAIL1)])
        pltpu.sync_copy(src_hbm.at[pl.ds(ebase + NFULL1 * CH, TAIL1)],
                        srcb.at[pl.ds(0, TAIL1)])

    ntailp = jnp.where(w < NW - 1, TAIL0 // 32, TAIL1 // 32)
    cnta, cntb = scan_pairs(ntailp, carry)

    neg1 = jnp.full((16,), -1, jnp.int32)

    def flush(lp, cnt, li):
        roundup = ((cnt + CH - 1) // CH) * CH

        def fill(j, _):
            lp[pl.ds(cnt + j * 16, 16)] = neg1
            return 0
        lax.fori_loop(0, (roundup - cnt + 15) // 16, fill, 0)
        pltpu.sync_copy(lp.at[pl.ds(0, CROW)],
                        cpk_hbm.at[pl.ds(li * CROW, CROW)])
        cstage[pl.ds(0, 16)] = jnp.zeros((16,), jnp.int32) + cnt
        pltpu.sync_copy(cstage, cnts_hbm.at[pl.ds(li * 16, 16)])

    flush(lpa, cnta, 2 * w)
    flush(lpb, cntb, 2 * w + 1)


def _prefilter(dst, src, tgt):
    return pl.kernel(
        _pre_body,
        out_type=[jax.ShapeDtypeStruct((NL * CROW,), jnp.int32),
                  jax.ShapeDtypeStruct((NL * 16,), jnp.int32)],
        mesh=_MESH,
        compiler_params=_SC_PARAMS,
        scratch_types=[pltpu.VMEM((NP,), jnp.int32),
                       pltpu.VMEM((U,), jnp.int32),
                       pltpu.VMEM((CH,), jnp.int32),
                       pltpu.VMEM((CH,), jnp.int32),
                       pltpu.VMEM((CBUF,), jnp.int32),
                       pltpu.VMEM((CBUF,), jnp.int32),
                       pltpu.VMEM((16,), jnp.int32)],
    )(dst, src, tgt)


def _merge_body(cpk_hbm, cnts_hbm, y_hbm, aggp_hbm,
                cntsv, pkb, lpk, stage_a, stage_b, rows_a, rows_b,
                acc, sem_a, sem_b):
    w = _wid()
    sgrp = w // NQ
    quarter = w % NQ
    lo = sgrp * SW2
    pltpu.sync_copy(cnts_hbm, cntsv)

    ninf = jnp.full((16,), -jnp.inf, jnp.float32)

    def init_acc(i, _):
        for j in range(8):
            acc[pl.ds(i * 128 + j * 16, 16)] = ninf
        return 0
    lax.fori_loop(0, (ACCW // 16 + 7) // 8, init_acc, 0)

    dummy = jnp.zeros((16,), jnp.int32) + ((lo + SW2) << 16)

    def issue(sgi, stage, buf, sem):
        for k in range(KDR // 16):
            pk = lpk[pl.ds(sgi * KDR + k * 16, 16)]
            stage[pl.ds(k * 16, 16)] = pk & 0xFFFF
        return pltpu.async_copy(y_hbm.at[stage], buf, sem)

    def accum(sgi, buf):
        for k in range(KDR // 16):
            pk = lpk[pl.ds(sgi * KDR + k * 16, 16)]
            for l in range(16):
                b = ((pk[l] >> 16) - lo) * AW
                for c in range(AW // 16):
                    av = acc[pl.ds(b + c * 16, 16)]
                    rv = buf[k * 16 + l, pl.ds(c * 16, 16)]
                    acc[pl.ds(b + c * 16, 16)] = jnp.maximum(av, rv)

    def wait_for(stage, buf, sem):
        pltpu.make_async_copy(y_hbm.at[stage], buf, sem).wait()

    def drain(n):
        roundup = ((n + KDR - 1) // KDR) * KDR

        def fill(j, _):
            lpk[pl.ds(n + j * 16, 16)] = dummy
            return 0
        lax.fori_loop(0, (roundup - n + 15) // 16, fill, 0)
        nsg = roundup // KDR

        @pl.when(nsg > 0)
        def _():
            issue(0, stage_a, rows_a, sem_a)

            def pair(p, _):
                @pl.when(2 * p + 1 < nsg)
                def _():
                    issue(2 * p + 1, stage_b, rows_b, sem_b)
                wait_for(stage_a, rows_a, sem_a)
                accum(2 * p, rows_a)

                @pl.when(2 * p + 2 < nsg)
                def _():
                    issue(2 * p + 2, stage_a, rows_a, sem_a)

                @pl.when(2 * p + 1 < nsg)
                def _():
                    wait_for(stage_b, rows_b, sem_b)
                    accum(2 * p + 1, rows_b)
                return 0
            lax.fori_loop(0, (nsg + 1) // 2, pair, 0)

    def one_list(j, cnt):
        v = quarter * LPQ + j
        nch = (cntsv[pl.ds(v * 16, 16)][0] + CH - 1) // CH

        def chunk(ci, cnt):
            pltpu.sync_copy(cpk_hbm.at[pl.ds(v * CROW + ci * CH, CH)], pkb)

            def grp(g, cnt):
                pk = pkb[pl.ds(g * 16, 16)]
                sl = pk >> 16
                m = (sl >= lo) & (sl < lo + SW2)
                plsc.store_compressed(lpk.at[pl.ds(cnt, 16)], pk, mask=m)
                cnt = cnt + plsc.all_reduce_population_count(m)[0]
                full = cnt > LCAP - 16

                @pl.when(full)
                def _():
                    drain(cnt)
                return jnp.where(full, 0, cnt)
            return lax.fori_loop(0, CH // 16, grp, cnt)
        return lax.fori_loop(0, nch, chunk, cnt)

    cnt = lax.fori_loop(0, LPQ, one_list, jnp.int32(0))
    drain(cnt)
    pltpu.sync_copy(acc.at[pl.ds(0, SW2 * AW)],
                    aggp_hbm.at[pl.ds((quarter * U + lo) * AW, SW2 * AW)])


def _merge(cpk, cnts, y):
    return pl.kernel(
        _merge_body,
        out_type=jax.ShapeDtypeStruct((NQ * U * AW,), jnp.float32),
        mesh=_MESH,
        compiler_params=_SC_PARAMS,
        scratch_types=[pltpu.VMEM((NL * 16,), jnp.int32),
                       pltpu.VMEM((CH,), jnp.int32),
                       pltpu.VMEM((LBUF,), jnp.int32),
                       pltpu.VMEM((KDR,), jnp.int32),
                       pltpu.VMEM((KDR,), jnp.int32),
                       pltpu.VMEM((KDR, DP), jnp.float32),
                       pltpu.VMEM((KDR, DP), jnp.float32),
                       pltpu.VMEM((ACCA,), jnp.float32),
                       pltpu.SemaphoreType.DMA,
                       pltpu.SemaphoreType.DMA],
    )(cpk, cnts, y)


def _heads_body(aggp_ref, wo_ref, bo_ref,
                wt1_ref, bt1_ref, wt2_ref, bt2_ref,
                wm1_ref, bm1_ref, wm2_ref, bm2_ref,
                ot_ref, om_ref):
    a = jnp.max(aggp_ref[...], axis=0)[:, :D]
    a = jnp.where(jnp.isfinite(a), a, 0.0)
    r = jnp.dot(a, wo_ref[...], preferred_element_type=jnp.float32) + bo_ref[...]

    def group(w1_ref, b1_ref, w2_ref, b2_ref, o_ref):
        for k in range(K):
            h = jnp.maximum(
                jnp.dot(r, w1_ref[k], preferred_element_type=jnp.float32)
                + b1_ref[...][k][None, :], 0.0)
            o_ref[k] = (jnp.dot(h, w2_ref[k], preferred_element_type=jnp.float32)
                        + b2_ref[...][k][None, :])

    group(wt1_ref, bt1_ref, wt2_ref, bt2_ref, ot_ref)
    group(wm1_ref, bm1_ref, wm2_ref, bm2_ref, om_ref)


def _heads(aggp3, W_out, b_out2, Wt1, bt1, Wt2p, bt2p, Wm1, bm1, Wm2p, bm2p):
    return pl.pallas_call(
        _heads_body,
        out_shape=[jax.ShapeDtypeStruct((K, U, 4), jnp.float32),
                   jax.ShapeDtypeStruct((K, U, 4), jnp.float32)],
    )(aggp3, W_out, b_out2, Wt1, bt1, Wt2p, bt2p, Wm1, bm1, Wm2p, bm2p)


def _out_body(ht_hbm, hm_hbm, tgt_hbm, rt_hbm, rm_hbm,
              htv, hmv, tgtv, dmap_v, ob):
    w = _wid()
    base = w * (K * T * 2 // NW)
    myk = base // (T * 2)
    pltpu.sync_copy(ht_hbm.at[pl.ds(myk * U * 4, U * 4)], htv)
    pltpu.sync_copy(hm_hbm.at[pl.ds(myk * U * 4, U * 4)], hmv)
    _local_dmap(tgt_hbm, tgtv, dmap_v)
    lane = lax.iota(jnp.int32, 16)

    def emit(src_v, tgt_off, out_hbm):
        def g_body(g, _):
            p = base + g * 16 + lane
            rr = (p >> 1) & (T - 1)
            j = p & 1
            t = plsc.load_gather(tgtv, [rr + tgt_off])
            sl = plsc.load_gather(dmap_v, [t])
            val = plsc.load_gather(src_v, [sl * 4 + j])
            ob[pl.ds(g * 16, 16)] = val
            return 0
        lax.fori_loop(0, (K * T * 2 // NW) // 16, g_body, 0)
        pltpu.sync_copy(ob, out_hbm.at[pl.ds(base, K * T * 2 // NW)])

    emit(htv, 0, rt_hbm)
    emit(hmv, T, rm_hbm)


def _out_gather(ht, hm, tgt):
    return pl.kernel(
        _out_body,
        out_type=[jax.ShapeDtypeStruct((K * T * 2,), jnp.float32),
                  jax.ShapeDtypeStruct((K * T * 2,), jnp.float32)],
        mesh=_MESH,
        compiler_params=_SC_PARAMS,
        scratch_types=[pltpu.VMEM((U * 4,), jnp.float32),
                       pltpu.VMEM((U * 4,), jnp.float32),
                       pltpu.VMEM((U,), jnp.int32),
                       pltpu.VMEM((NP,), jnp.int32),
                       pltpu.VMEM((K * T * 2 // NW,), jnp.float32)],
    )(ht, hm, tgt)


def kernel(x, edge_index, target_tasklets, target_map_entry,
           W_msg, W_out, b_out, Wt1, bt1, Wt2, bt2, Wm1, bm1, Wm2, bm2):
    f32 = jnp.float32
    w_p = jnp.zeros((D, DP), f32).at[:, :D].set(W_msg)
    y = _msg_matmul(x, w_p)

    tgt = jnp.concatenate([target_tasklets[0], target_map_entry[0]])

    cpk, cnts = _prefilter(edge_index[1], edge_index[0], tgt)
    aggp_flat = _merge(cpk, cnts, y)
    aggp3 = aggp_flat.reshape(NQ, U, AW)

    b_out2 = b_out.reshape(1, D)
    Wt2p = jnp.zeros((K, D, 4), f32).at[:, :, :2].set(Wt2)
    bt2p = jnp.zeros((K, 4), f32).at[:, :2].set(bt2)
    Wm2p = jnp.zeros((K, D, 4), f32).at[:, :, :2].set(Wm2)
    bm2p = jnp.zeros((K, 4), f32).at[:, :2].set(bm2)
    ht, hm = _heads(aggp3, W_out, b_out2, Wt1, bt1, Wt2p, bt2p,
                    Wm1, bm1, Wm2p, bm2p)

    rt_flat, rm_flat = _out_gather(ht.reshape(K * U * 4), hm.reshape(K * U * 4),
                                   tgt)
    return rt_flat.reshape(K, T, 2), rm_flat.reshape(K, T, 2)


# speedup vs baseline: 13.0457x; 1.0046x over previous
import jax
import jax.numpy as jnp
from jax import lax
from jax.experimental import pallas as pl
from jax.experimental.pallas import tpu as pltpu
from jax.experimental.pallas import tpu_sc as plsc

N = 50000
E = 800000
D = 70
K = 4
T = 2048

NC, NS = 2, 16
NW = NC * NS
DP = 128
AW = 80
SN = 1568
NP = NW * SN
U = 2 * T
CH = 2048

EW1 = 25024
NFULL0, TAIL0 = 12, 448
NFULL1, TAIL1 = 11, 1728
CROW = 7 * CH
CBUF = CROW + 16
NL = 2 * NW

NSG, NQ = 4, 8
SW2 = U // NSG
LPQ = NL // NQ

LCAP = 2048
KDR = 64
LBUF = LCAP + KDR + 16
ACCW = (SW2 + 1) * AW
ACCA = ((ACCW // 16 + 7) // 8) * 128 + 16

_MESH = plsc.VectorSubcoreMesh(core_axis_name="c", subcore_axis_name="s",
                               num_cores=NC, num_subcores=NS)
_SC_PARAMS = pltpu.CompilerParams(needs_layout_passes=False)


def _wid():
    return lax.axis_index("s") * NC + lax.axis_index("c")


def _mm_body(x_ref, w_ref, o_ref):
    o_ref[...] = jnp.dot(x_ref[...], w_ref[...],
                         preferred_element_type=jnp.float32)


def _msg_matmul(x, w_p):
    BM = 512
    return pl.pallas_call(
        _mm_body,
        grid=(pl.cdiv(N, BM),),
        in_specs=[pl.BlockSpec((BM, D), lambda i: (i, 0)),
                  pl.BlockSpec((D, DP), lambda i: (0, 0))],
        out_specs=pl.BlockSpec((BM, DP), lambda i: (i, 0)),
        out_shape=jax.ShapeDtypeStruct((N, DP), jnp.float32),
    )(x, w_p)


def _local_dmap(tgt_hbm, tgtv, dmap_v):
    neg1 = jnp.full((16,), -1, jnp.int32)

    def init(i, _):
        for j in range(8):
            dmap_v[pl.ds(i * 128 + j * 16, 16)] = neg1
        return 0
    lax.fori_loop(0, NP // 128, init, 0)

    pltpu.sync_copy(tgt_hbm, tgtv)
    lane = lax.iota(jnp.int32, 16)

    def scat(g, _):
        t = tgtv[pl.ds(g * 16, 16)]
        plsc.store_scatter(dmap_v, [t], g * 16 + lane)
        return 0
    lax.fori_loop(0, U // 16, scat, 0)


def _pre_body(dst_hbm, src_hbm, tgt_hbm, cpk_hbm, cnts_hbm,
              dmap_v, tgtv, dstb, srcb, dstb2, srcb2, lpa, lpb, cstage,
              semd, sems, semd2, sems2):
    w = _wid()
    ebase = w * EW1
    nfull = jnp.where(w < NW - 1, NFULL0, NFULL1)

    def issue_chunk(ci, db, sb, sd, ss):
        pltpu.async_copy(dst_hbm.at[pl.ds(ebase + ci * CH, CH)], db, sd)
        pltpu.async_copy(src_hbm.at[pl.ds(ebase + ci * CH, CH)], sb, ss)

    def wait_chunk(db, sb, sd, ss):
        pltpu.make_async_copy(dst_hbm.at[pl.ds(ebase, CH)], db, sd).wait()
        pltpu.make_async_copy(src_hbm.at[pl.ds(ebase, CH)], sb, ss).wait()

    @pl.when(nfull > 0)
    def _():
        issue_chunk(0, dstb, srcb, semd, sems)
    _local_dmap(tgt_hbm, tgtv, dmap_v)

    def scan_pairs(npair, db, sb, carry):
        cnta, cntb = carry

        def pair(g, carry):
            cnta, cntb = carry
            dva = db[pl.ds(g * 32, 16)]
            dvb = db[pl.ds(g * 32 + 16, 16)]
            sla = plsc.load_gather(dmap_v, [dva])
            slb = plsc.load_gather(dmap_v, [dvb])
            ma = sla >= 0
            mb = slb >= 0
            pka = (sla << 16) | sb[pl.ds(g * 32, 16)]
            pkb2 = (slb << 16) | sb[pl.ds(g * 32 + 16, 16)]
            plsc.store_compressed(lpa.at[pl.ds(cnta, 16)], pka, mask=ma)
            plsc.store_compressed(lpb.at[pl.ds(cntb, 16)], pkb2, mask=mb)
            return (cnta + plsc.all_reduce_population_count(ma)[0],
                    cntb + plsc.all_reduce_population_count(mb)[0])
        return lax.fori_loop(0, npair, pair, (cnta, cntb))

    def cpair(p, carry):
        @pl.when(2 * p + 1 < nfull)
        def _():
            issue_chunk(2 * p + 1, dstb2, srcb2, semd2, sems2)
        wait_chunk(dstb, srcb, semd, sems)
        carry = scan_pairs(CH // 32, dstb, srcb, carry)

        @pl.when(2 * p + 2 < nfull)
        def _():
            issue_chunk(2 * p + 2, dstb, srcb, semd, sems)

        def do_b(carry):
            wait_chunk(dstb2, srcb2, semd2, sems2)
            return scan_pairs(CH // 32, dstb2, srcb2, carry)
        return lax.cond(2 * p + 1 < nfull, do_b, lambda c: c, carry)

    carry = lax.fori_loop(0, (nfull + 1) // 2, cpair,
                          (jnp.int32(0), jnp.int32(0)))

    @pl.when(w < NW - 1)
    def _():
        pltpu.sync_copy(dst_hbm.at[pl.ds(ebase + NFULL0 * CH, TAIL0)],
                        dstb.at[pl.ds(0, TAIL0)])
        pltpu.sync_copy(src_hbm.at[pl.ds(ebase + NFULL0 * CH, TAIL0)],
                        srcb.at[pl.ds(0, TAIL0)])

    @pl.when(w == NW - 1)
    def _():
        pltpu.sync_copy(dst_hbm.at[pl.ds(ebase + NFULL1 * CH, TAIL1)],
                        dstb.at[pl.ds(0, TAIL1)])
        pltpu.sync_copy(src_hbm.at[pl.ds(ebase + NFULL1 * CH, TAIL1)],
                        srcb.at[pl.ds(0, TAIL1)])

    ntailp = jnp.where(w < NW - 1, TAIL0 // 32, TAIL1 // 32)
    cnta, cntb = scan_pairs(ntailp, dstb, srcb, carry)

    neg1 = jnp.full((16,), -1, jnp.int32)

    def flush(lp, cnt, li):
        roundup = ((cnt + CH - 1) // CH) * CH

        def fill(j, _):
            lp[pl.ds(cnt + j * 16, 16)] = neg1
            return 0
        lax.fori_loop(0, (roundup - cnt + 15) // 16, fill, 0)
        pltpu.sync_copy(lp.at[pl.ds(0, CROW)],
                        cpk_hbm.at[pl.ds(li * CROW, CROW)])
        cstage[pl.ds(0, 16)] = jnp.zeros((16,), jnp.int32) + cnt
        pltpu.sync_copy(cstage, cnts_hbm.at[pl.ds(li * 16, 16)])

    flush(lpa, cnta, 2 * w)
    flush(lpb, cntb, 2 * w + 1)


def _prefilter(dst, src, tgt):
    return pl.kernel(
        _pre_body,
        out_type=[jax.ShapeDtypeStruct((NL * CROW,), jnp.int32),
                  jax.ShapeDtypeStruct((NL * 16,), jnp.int32)],
        mesh=_MESH,
        compiler_params=_SC_PARAMS,
        scratch_types=[pltpu.VMEM((NP,), jnp.int32),
                       pltpu.VMEM((U,), jnp.int32),
                       pltpu.VMEM((CH,), jnp.int32),
                       pltpu.VMEM((CH,), jnp.int32),
                       pltpu.VMEM((CH,), jnp.int32),
                       pltpu.VMEM((CH,), jnp.int32),
                       pltpu.VMEM((CBUF,), jnp.int32),
                       pltpu.VMEM((CBUF,), jnp.int32),
                       pltpu.VMEM((16,), jnp.int32),
                       pltpu.SemaphoreType.DMA,
                       pltpu.SemaphoreType.DMA,
                       pltpu.SemaphoreType.DMA,
                       pltpu.SemaphoreType.DMA],
    )(dst, src, tgt)


def _merge_body(cpk_hbm, cnts_hbm, y_hbm, aggp_hbm,
                cntsv, pkb, lpk, stage_a, stage_b, rows_a, rows_b,
                acc, sem_a, sem_b):
    w = _wid()
    sgrp = w // NQ
    quarter = w % NQ
    lo = sgrp * SW2
    pltpu.sync_copy(cnts_hbm, cntsv)

    ninf = jnp.full((16,), -jnp.inf, jnp.float32)

    def init_acc(i, _):
        for j in range(8):
            acc[pl.ds(i * 128 + j * 16, 16)] = ninf
        return 0
    lax.fori_loop(0, (ACCW // 16 + 7) // 8, init_acc, 0)

    dummy = jnp.zeros((16,), jnp.int32) + ((lo + SW2) << 16)

    def issue(sgi, stage, buf, sem):
        for k in range(KDR // 16):
            pk = lpk[pl.ds(sgi * KDR + k * 16, 16)]
            stage[pl.ds(k * 16, 16)] = pk & 0xFFFF
        return pltpu.async_copy(y_hbm.at[stage], buf, sem)

    def accum(sgi, buf):
        for k in range(KDR // 16):
            pk = lpk[pl.ds(sgi * KDR + k * 16, 16)]
            for l in range(16):
                b = ((pk[l] >> 16) - lo) * AW
                for c in range(AW // 16):
                    av = acc[pl.ds(b + c * 16, 16)]
                    rv = buf[k * 16 + l, pl.ds(c * 16, 16)]
                    acc[pl.ds(b + c * 16, 16)] = jnp.maximum(av, rv)

    def wait_for(stage, buf, sem):
        pltpu.make_async_copy(y_hbm.at[stage], buf, sem).wait()

    def drain(n):
        roundup = ((n + KDR - 1) // KDR) * KDR

        def fill(j, _):
            lpk[pl.ds(n + j * 16, 16)] = dummy
            return 0
        lax.fori_loop(0, (roundup - n + 15) // 16, fill, 0)
        nsg = roundup // KDR

        @pl.when(nsg > 0)
        def _():
            issue(0, stage_a, rows_a, sem_a)

            def pair(p, _):
                @pl.when(2 * p + 1 < nsg)
                def _():
                    issue(2 * p + 1, stage_b, rows_b, sem_b)
                wait_for(stage_a, rows_a, sem_a)
                accum(2 * p, rows_a)

                @pl.when(2 * p + 2 < nsg)
                def _():
                    issue(2 * p + 2, stage_a, rows_a, sem_a)

                @pl.when(2 * p + 1 < nsg)
                def _():
                    wait_for(stage_b, rows_b, sem_b)
                    accum(2 * p + 1, rows_b)
                return 0
            lax.fori_loop(0, (nsg + 1) // 2, pair, 0)

    def one_list(j, cnt):
        v = quarter * LPQ + j
        nch = (cntsv[pl.ds(v * 16, 16)][0] + CH - 1) // CH

        def chunk(ci, cnt):
            pltpu.sync_copy(cpk_hbm.at[pl.ds(v * CROW + ci * CH, CH)], pkb)

            def grp(g, cnt):
                pk = pkb[pl.ds(g * 16, 16)]
                sl = pk >> 16
                m = (sl >= lo) & (sl < lo + SW2)
                plsc.store_compressed(lpk.at[pl.ds(cnt, 16)], pk, mask=m)
                cnt = cnt + plsc.all_reduce_population_count(m)[0]
                full = cnt > LCAP - 16

                @pl.when(full)
                def _():
                    drain(cnt)
                return jnp.where(full, 0, cnt)
            return lax.fori_loop(0, CH // 16, grp, cnt)
        return lax.fori_loop(0, nch, chunk, cnt)

    cnt = lax.fori_loop(0, LPQ, one_list, jnp.int32(0))
    drain(cnt)
    pltpu.sync_copy(acc.at[pl.ds(0, SW2 * AW)],
                    aggp_hbm.at[pl.ds((quarter * U + lo) * AW, SW2 * AW)])


def _merge(cpk, cnts, y):
    return pl.kernel(
        _merge_body,
        out_type=jax.ShapeDtypeStruct((NQ * U * AW,), jnp.float32),
        mesh=_MESH,
        compiler_params=_SC_PARAMS,
        scratch_types=[pltpu.VMEM((NL * 16,), jnp.int32),
                       pltpu.VMEM((CH,), jnp.int32),
                       pltpu.VMEM((LBUF,), jnp.int32),
                       pltpu.VMEM((KDR,), jnp.int32),
                       pltpu.VMEM((KDR,), jnp.int32),
                       pltpu.VMEM((KDR, DP), jnp.float32),
                       pltpu.VMEM((KDR, DP), jnp.float32),
                       pltpu.VMEM((ACCA,), jnp.float32),
                       pltpu.SemaphoreType.DMA,
                       pltpu.SemaphoreType.DMA],
    )(cpk, cnts, y)


def _heads_body(aggp_ref, wo_ref, bo_ref,
                wt1_ref, bt1_ref, wt2_ref, bt2_ref,
                wm1_ref, bm1_ref, wm2_ref, bm2_ref,
                ot_ref, om_ref):
    a = jnp.max(aggp_ref[...], axis=0)[:, :D]
    a = jnp.where(jnp.isfinite(a), a, 0.0)
    r = jnp.dot(a, wo_ref[...], preferred_element_type=jnp.float32) + bo_ref[...]

    def group(w1_ref, b1_ref, w2_ref, b2_ref, o_ref):
        for k in range(K):
            h = jnp.maximum(
                jnp.dot(r, w1_ref[k], preferred_element_type=jnp.float32)
                + b1_ref[...][k][None, :], 0.0)
            o_ref[k] = (jnp.dot(h, w2_ref[k], preferred_element_type=jnp.float32)
                        + b2_ref[...][k][None, :])

    group(wt1_ref, bt1_ref, wt2_ref, bt2_ref, ot_ref)
    group(wm1_ref, bm1_ref, wm2_ref, bm2_ref, om_ref)


def _heads(aggp3, W_out, b_out2, Wt1, bt1, Wt2p, bt2p, Wm1, bm1, Wm2p, bm2p):
    return pl.pallas_call(
        _heads_body,
        out_shape=[jax.ShapeDtypeStruct((K, U, 4), jnp.float32),
                   jax.ShapeDtypeStruct((K, U, 4), jnp.float32)],
    )(aggp3, W_out, b_out2, Wt1, bt1, Wt2p, bt2p, Wm1, bm1, Wm2p, bm2p)


def _out_body(ht_hbm, hm_hbm, tgt_hbm, rt_hbm, rm_hbm,
              htv, hmv, tgtv, dmap_v, ob):
    w = _wid()
    base = w * (K * T * 2 // NW)
    myk = base // (T * 2)
    pltpu.sync_copy(ht_hbm.at[pl.ds(myk * U * 4, U * 4)], htv)
    pltpu.sync_copy(hm_hbm.at[pl.ds(myk * U * 4, U * 4)], hmv)
    _local_dmap(tgt_hbm, tgtv, dmap_v)
    lane = lax.iota(jnp.int32, 16)

    def emit(src_v, tgt_off, out_hbm):
        def g_body(g, _):
            p = base + g * 16 + lane
            rr = (p >> 1) & (T - 1)
            j = p & 1
            t = plsc.load_gather(tgtv, [rr + tgt_off])
            sl = plsc.load_gather(dmap_v, [t])
            val = plsc.load_gather(src_v, [sl * 4 + j])
            ob[pl.ds(g * 16, 16)] = val
            return 0
        lax.fori_loop(0, (K * T * 2 // NW) // 16, g_body, 0)
        pltpu.sync_copy(ob, out_hbm.at[pl.ds(base, K * T * 2 // NW)])

    emit(htv, 0, rt_hbm)
    emit(hmv, T, rm_hbm)


def _out_gather(ht, hm, tgt):
    return pl.kernel(
        _out_body,
        out_type=[jax.ShapeDtypeStruct((K * T * 2,), jnp.float32),
                  jax.ShapeDtypeStruct((K * T * 2,), jnp.float32)],
        mesh=_MESH,
        compiler_params=_SC_PARAMS,
        scratch_types=[pltpu.VMEM((U * 4,), jnp.float32),
                       pltpu.VMEM((U * 4,), jnp.float32),
                       pltpu.VMEM((U,), jnp.int32),
                       pltpu.VMEM((NP,), jnp.int32),
                       pltpu.VMEM((K * T * 2 // NW,), jnp.float32)],
    )(ht, hm, tgt)


def kernel(x, edge_index, target_tasklets, target_map_entry,
           W_msg, W_out, b_out, Wt1, bt1, Wt2, bt2, Wm1, bm1, Wm2, bm2):
    f32 = jnp.float32
    w_p = jnp.zeros((D, DP), f32).at[:, :D].set(W_msg)
    y = _msg_matmul(x, w_p)

    tgt = jnp.concatenate([target_tasklets[0], target_map_entry[0]])

    cpk, cnts = _prefilter(edge_index[1], edge_index[0], tgt)
    aggp_flat = _merge(cpk, cnts, y)
    aggp3 = aggp_flat.reshape(NQ, U, AW)

    b_out2 = b_out.reshape(1, D)
    Wt2p = jnp.zeros((K, D, 4), f32).at[:, :, :2].set(Wt2)
    bt2p = jnp.zeros((K, 4), f32).at[:, :2].set(bt2)
    Wm2p = jnp.zeros((K, D, 4), f32).at[:, :, :2].set(Wm2)
    bm2p = jnp.zeros((K, 4), f32).at[:, :2].set(bm2)
    ht, hm = _heads(aggp3, W_out, b_out2, Wt1, bt1, Wt2p, bt2p,
                    Wm1, bm1, Wm2p, bm2p)

    rt_flat, rm_flat = _out_gather(ht.reshape(K * U * 4), hm.reshape(K * U * 4),
                                   tgt)
    return rt_flat.reshape(K, T, 2), rm_flat.reshape(K, T, 2)
